# baseline (speedup 1.0000x reference)
import math

import jax
import jax.numpy as jnp
from jax.experimental import pallas as pl
from jax.experimental.pallas import tpu as pltpu

D_MODEL = 256
GRID_DIM = 30
G = 9
GPB = 1


def _body(x_ref, io_ref, pair_ref, o_ref, comb_scr):
    b = pl.program_id(0)
    j = pl.program_id(1)

    @pl.when(jnp.logical_and(b == 0, j == 0))
    def _init():
        dim = D_MODEL // 2
        h = jax.lax.broadcasted_iota(jnp.int32, (GRID_DIM, GRID_DIM, 2 * dim), 0)
        w = jax.lax.broadcasted_iota(jnp.int32, (GRID_DIM, GRID_DIM, 2 * dim), 1)
        c = jax.lax.broadcasted_iota(jnp.int32, (GRID_DIM, GRID_DIM, 2 * dim), 2)
        pos = jnp.where(c < dim, h, w).astype(jnp.float32)
        cl = c % dim
        freq = jnp.exp((cl - cl % 2).astype(jnp.float32) * (-math.log(10000.0) / dim))
        angle = pos * freq
        pos_emb = jnp.where(cl % 2 == 0, jnp.sin(angle), jnp.cos(angle))
        for gg in range(G):
            comb_scr[gg, :, :, 0:256] = pos_emb
            comb_scr[gg, :, :, 256:320] = jnp.broadcast_to(
                io_ref[gg % 2, :][None, None, :], (GRID_DIM, GRID_DIM, 64))
            comb_scr[gg, :, :, 320:384] = jnp.broadcast_to(
                pair_ref[gg // 2, :][None, None, :], (GRID_DIM, GRID_DIM, 64))

    o_ref[0, 0] = x_ref[0, 0] + comb_scr[j]


@jax.jit
def kernel(x, io_table, pair_table):
    B, Gd, H, W, C = x.shape
    return pl.pallas_call(
        _body,
        grid=(B, Gd // GPB),
        in_specs=[
            pl.BlockSpec((1, GPB, H, W, C), lambda b, j: (b, j, 0, 0, 0)),
            pl.BlockSpec(memory_space=pltpu.VMEM),
            pl.BlockSpec(memory_space=pltpu.VMEM),
        ],
        out_specs=pl.BlockSpec((1, GPB, H, W, C), lambda b, j: (b, j, 0, 0, 0)),
        out_shape=jax.ShapeDtypeStruct((B, Gd, H, W, C), x.dtype),
        scratch_shapes=[
            pltpu.VMEM((G, H, W, C), jnp.float32),
        ],
    )(x, io_table, pair_table)


# speedup vs baseline: 1.0785x; 1.0785x over previous
import math

import jax
import jax.numpy as jnp
from jax.experimental import pallas as pl
from jax.experimental.pallas import tpu as pltpu

D_MODEL = 256
GRID_DIM = 30
G = 9
GPB = 3
BPB = 2


def _emb_body(io_ref, pair_ref, comb_ref):
    dim = D_MODEL // 2
    h = jax.lax.broadcasted_iota(jnp.int32, (GRID_DIM, GRID_DIM, 2 * dim), 0)
    w = jax.lax.broadcasted_iota(jnp.int32, (GRID_DIM, GRID_DIM, 2 * dim), 1)
    c = jax.lax.broadcasted_iota(jnp.int32, (GRID_DIM, GRID_DIM, 2 * dim), 2)
    pos = jnp.where(c < dim, h, w).astype(jnp.float32)
    cl = c % dim
    freq = jnp.exp((cl - cl % 2).astype(jnp.float32) * (-math.log(10000.0) / dim))
    angle = pos * freq
    pos_emb = jnp.where(cl % 2 == 0, jnp.sin(angle), jnp.cos(angle))
    for gg in range(G):
        comb_ref[gg, :, :, 0:256] = pos_emb
        comb_ref[gg, :, :, 256:320] = jnp.broadcast_to(
            io_ref[gg % 2, :][None, None, :], (GRID_DIM, GRID_DIM, 64))
        comb_ref[gg, :, :, 320:384] = jnp.broadcast_to(
            pair_ref[gg // 2, :][None, None, :], (GRID_DIM, GRID_DIM, 64))


def _add_body(x_ref, c_ref, o_ref):
    o_ref[...] = x_ref[...] + c_ref[None]


@jax.jit
def kernel(x, io_table, pair_table):
    B, Gd, H, W, C = x.shape
    comb = pl.pallas_call(
        _emb_body,
        in_specs=[
            pl.BlockSpec(memory_space=pltpu.VMEM),
            pl.BlockSpec(memory_space=pltpu.VMEM),
        ],
        out_specs=pl.BlockSpec((Gd, H, W, C), lambda: (0, 0, 0, 0)),
        out_shape=jax.ShapeDtypeStruct((Gd, H, W, C), x.dtype),
    )(io_table, pair_table)
    return pl.pallas_call(
        _add_body,
        grid=(Gd // GPB, B // BPB),
        in_specs=[
            pl.BlockSpec((BPB, GPB, H, W, C), lambda j, b: (b, j, 0, 0, 0)),
            pl.BlockSpec((GPB, H, W, C), lambda j, b: (j, 0, 0, 0)),
        ],
        out_specs=pl.BlockSpec((BPB, GPB, H, W, C), lambda j, b: (b, j, 0, 0, 0)),
        out_shape=jax.ShapeDtypeStruct((B, Gd, H, W, C), x.dtype),
    )(x, comb)
